# Initial kernel scaffold; baseline (speedup 1.0000x reference)
import functools

import jax
import jax.numpy as jnp
from jax import lax
from jax.experimental import pallas as pl
from jax.experimental.pallas import tpu as pltpu
from jax.experimental.pallas import tpu_sc as plsc

_NC = 2
_NS = 16
_NW = _NC * _NS
_L = 16
_CH = 128


def _softplus(t):
    return jnp.maximum(t, 0.0) + jnp.log(1.0 + jnp.exp(-jnp.abs(t)))


def _proj_body(x_ref, ks_ref, kd_ref, st_ref, kst_ref, b_ref,
               tsrc_ref, tdst_ref, s32_ref):
    x = x_ref[...]
    tsrc_ref[...] = jnp.dot(x, ks_ref[...], preferred_element_type=jnp.float32)
    tdst_ref[...] = jnp.dot(x, kd_ref[...], preferred_element_type=jnp.float32)

    @pl.when(pl.program_id(0) == 0)
    def _():
        s32_ref[...] = (
            jnp.dot(st_ref[...], kst_ref[...],
                    preferred_element_type=jnp.float32)
            + b_ref[...]
        )


def _gather_body(tsrc_hbm, tdst_hbm, src_hbm, dst_hbm, pre_hbm,
                 isrc_v, idst_v, ra_v, rb_v, sem_a, sem_b):
    e = src_hbm.shape[0]
    nch = e // _CH
    cpw = nch // _NW
    extra = nch - _NW * cpw
    dcat = ra_v.shape[1]
    wid = lax.axis_index("s") * _NC + lax.axis_index("c")

    def do_chunk(cidx):
        base = cidx * _CH
        pltpu.sync_copy(src_hbm.at[pl.ds(base, _CH)], isrc_v)
        pltpu.sync_copy(dst_hbm.at[pl.ds(base, _CH)], idst_v)
        cp_a = pltpu.async_copy(tsrc_hbm.at[isrc_v], ra_v, sem_a)
        cp_b = pltpu.async_copy(tdst_hbm.at[idst_v], rb_v, sem_b)
        cp_a.wait()
        cp_b.wait()

        def add_row(r, carry):
            for j in range(dcat // _L):
                sl = pl.ds(j * _L, _L)
                ra_v[r, sl] = ra_v[r, sl] + rb_v[r, sl]
            return carry

        lax.fori_loop(0, _CH, add_row, 0)
        pltpu.sync_copy(ra_v, pre_hbm.at[pl.ds(base, _CH)])

    def body(i, carry):
        do_chunk(wid * cpw + i)
        return carry

    lax.fori_loop(0, cpw, body, 0)

    @pl.when(wid < extra)
    def _():
        do_chunk(_NW * cpw + wid)


def _edge_body(pre_ref, bond_ref, bg_ref, s32_ref, kb_ref, out_ref):
    pre = pre_ref[...]
    bond = bond_ref[...]
    row = bg_ref[0]
    ng = s32_ref.shape[0]
    onehot_t = (lax.broadcasted_iota(jnp.int32, (ng, row.shape[1]), 0)
                == row).astype(jnp.float32)
    contrib = lax.dot_general(onehot_t, s32_ref[...],
                              (((0,), (0,)), ((), ())),
                              preferred_element_type=jnp.float32)
    t = pre + jnp.dot(bond, kb_ref[...],
                      preferred_element_type=jnp.float32) + contrib
    d = out_ref.shape[1]
    ts = t[:, :d]
    tg = t[:, d:]
    sig = 1.0 / (1.0 + jnp.exp(-ts))
    out_ref[...] = sig * _softplus(tg)


def _scatter_body(trans_hbm, src_hbm, agg_hbm, acc_sh, vbuf, ibuf, zbuf):
    e = src_hbm.shape[0]
    n = acc_sh.shape[0]
    nch = e // _CH
    cpw = nch // _NW
    extra = nch - _NW * cpw
    rpt = n // _NS
    zr = zbuf.shape[0]
    nz = rpt // zr
    da = acc_sh.shape[1]
    cid = lax.axis_index("c")
    sid = lax.axis_index("s")
    wid = sid * _NC + cid

    def zero_row(r, carry):
        for j in range(da // _L):
            zbuf[r, pl.ds(j * _L, _L)] = jnp.zeros((_L,), jnp.float32)
        return carry

    lax.fori_loop(0, zr, zero_row, 0)
    for k in range(nz):
        pltpu.sync_copy(zbuf, acc_sh.at[pl.ds(sid * rpt + k * zr, zr)])
    plsc.subcore_barrier()

    def do_chunk(cidx):
        base = cidx * _CH
        pltpu.sync_copy(src_hbm.at[pl.ds(base, _CH)], ibuf)
        pltpu.sync_copy(trans_hbm.at[pl.ds(base, _CH)], vbuf)
        pltpu.sync_copy(vbuf, acc_sh.at[ibuf], add=True)

    def body(i, carry):
        do_chunk(wid * cpw + i)
        return carry

    lax.fori_loop(0, cpw, body, 0)

    @pl.when(wid < extra)
    def _():
        do_chunk(_NW * cpw + wid)

    plsc.subcore_barrier()
    pltpu.sync_copy(acc_sh.at[pl.ds(sid * rpt, rpt)],
                    agg_hbm.at[pl.ds(cid * n + sid * rpt, rpt)])


def _out_body(x_ref, a0_ref, a1_ref, out_ref):
    t = x_ref[...] + a0_ref[...] + a1_ref[...]
    out_ref[...] = _softplus(t)


def kernel(atom_features, bond_features, state_attrs, pair_indices,
           atom_graph_indices, bond_graph_indices,
           kernel_s, bias_s, kernel_g, bias_g):
    del atom_graph_indices
    n, da = atom_features.shape
    e, de = bond_features.shape
    ng, dst_dim = state_attrs.shape
    dcat = 2 * da

    kk = jnp.concatenate([kernel_s, kernel_g], axis=1)
    k_src = kk[:da]
    k_dst = kk[da:2 * da]
    k_state = kk[2 * da:2 * da + dst_dim]
    k_bond = kk[2 * da + dst_dim:]
    bias = jnp.concatenate([bias_s, bias_g]).reshape(1, dcat)

    src = pair_indices[:, 0]
    dst = pair_indices[:, 1]

    nb = 8
    bn = n // nb
    tsrc, tdst, s32 = pl.pallas_call(
        _proj_body,
        grid=(nb,),
        in_specs=[
            pl.BlockSpec((bn, da), lambda i: (i, 0)),
            pl.BlockSpec((da, dcat), lambda i: (0, 0)),
            pl.BlockSpec((da, dcat), lambda i: (0, 0)),
            pl.BlockSpec((ng, dst_dim), lambda i: (0, 0)),
            pl.BlockSpec((dst_dim, dcat), lambda i: (0, 0)),
            pl.BlockSpec((1, dcat), lambda i: (0, 0)),
        ],
        out_specs=[
            pl.BlockSpec((bn, dcat), lambda i: (i, 0)),
            pl.BlockSpec((bn, dcat), lambda i: (i, 0)),
            pl.BlockSpec((ng, dcat), lambda i: (0, 0)),
        ],
        out_shape=[
            jax.ShapeDtypeStruct((n, dcat), jnp.float32),
            jax.ShapeDtypeStruct((n, dcat), jnp.float32),
            jax.ShapeDtypeStruct((ng, dcat), jnp.float32),
        ],
    )(atom_features, k_src, k_dst, state_attrs, k_state, bias)

    mesh = plsc.VectorSubcoreMesh(core_axis_name="c", subcore_axis_name="s")
    pre = pl.kernel(
        _gather_body,
        mesh=mesh,
        out_type=jax.ShapeDtypeStruct((e, dcat), jnp.float32),
        scratch_types=[
            pltpu.VMEM((_CH,), jnp.int32),
            pltpu.VMEM((_CH,), jnp.int32),
            pltpu.VMEM((_CH, dcat), jnp.float32),
            pltpu.VMEM((_CH, dcat), jnp.float32),
            pltpu.SemaphoreType.DMA,
            pltpu.SemaphoreType.DMA,
        ],
    )(tsrc, tdst, src, dst)

    be = 512
    nbe = e // be
    bg3 = bond_graph_indices.reshape(nbe, 1, be)
    transformed = pl.pallas_call(
        _edge_body,
        grid=(nbe,),
        in_specs=[
            pl.BlockSpec((be, dcat), lambda i: (i, 0)),
            pl.BlockSpec((be, de), lambda i: (i, 0)),
            pl.BlockSpec((1, 1, be), lambda i: (i, 0, 0)),
            pl.BlockSpec((ng, dcat), lambda i: (0, 0)),
            pl.BlockSpec((de, dcat), lambda i: (0, 0)),
        ],
        out_specs=pl.BlockSpec((be, da), lambda i: (i, 0)),
        out_shape=jax.ShapeDtypeStruct((e, da), jnp.float32),
    )(pre, bond_features, bg3, s32, k_bond)

    agg = pl.kernel(
        _scatter_body,
        mesh=mesh,
        out_type=jax.ShapeDtypeStruct((_NC * n, da), jnp.float32),
        scratch_types=[
            pltpu.VMEM_SHARED((n, da), jnp.float32),
            pltpu.VMEM((_CH, da), jnp.float32),
            pltpu.VMEM((_CH,), jnp.int32),
            pltpu.VMEM((125, da), jnp.float32),
        ],
    )(transformed, src)

    out = pl.pallas_call(
        _out_body,
        grid=(nb,),
        in_specs=[
            pl.BlockSpec((bn, da), lambda i: (i, 0)),
            pl.BlockSpec((bn, da), lambda i: (i, 0)),
            pl.BlockSpec((bn, da), lambda i: (i + nb, 0)),
        ],
        out_specs=pl.BlockSpec((bn, da), lambda i: (i, 0)),
        out_shape=jax.ShapeDtypeStruct((n, da), jnp.float32),
    )(atom_features, agg, agg)
    return out


# speedup vs baseline: 2.7221x; 2.7221x over previous
import functools

import jax
import jax.numpy as jnp
from jax import lax
from jax.experimental import pallas as pl
from jax.experimental.pallas import tpu as pltpu
from jax.experimental.pallas import tpu_sc as plsc

_NC = 2
_NS = 16
_NW = _NC * _NS
_L = 16
_CH = 128


def _softplus(t):
    return jnp.maximum(t, 0.0) + jnp.log(1.0 + jnp.exp(-jnp.abs(t)))


def _proj_body(x_ref, ks_ref, kd_ref, st_ref, kst_ref, b_ref,
               tsrc_ref, tdst_ref, s32_ref):
    x = x_ref[...]
    tsrc_ref[...] = jnp.dot(x, ks_ref[...], preferred_element_type=jnp.float32)
    tdst_ref[...] = jnp.dot(x, kd_ref[...], preferred_element_type=jnp.float32)

    @pl.when(pl.program_id(0) == 0)
    def _():
        s32_ref[...] = (
            jnp.dot(st_ref[...], kst_ref[...],
                    preferred_element_type=jnp.float32)
            + b_ref[...]
        )


def _gather_body(tsrc_hbm, tdst_hbm, src_hbm, dst_hbm, pre_hbm,
                 isrc_v, idst_v, ra_v, rb_v, sem_a, sem_b):
    e = src_hbm.shape[0]
    nch = e // _CH
    cpw = nch // _NW
    extra = nch - _NW * cpw
    dcat = ra_v.shape[1]
    wid = lax.axis_index("s") * _NC + lax.axis_index("c")

    def do_chunk(cidx):
        base = cidx * _CH
        pltpu.sync_copy(src_hbm.at[pl.ds(base, _CH)], isrc_v)
        pltpu.sync_copy(dst_hbm.at[pl.ds(base, _CH)], idst_v)
        cp_a = pltpu.async_copy(tsrc_hbm.at[isrc_v], ra_v, sem_a)
        cp_b = pltpu.async_copy(tdst_hbm.at[idst_v], rb_v, sem_b)
        cp_a.wait()
        cp_b.wait()

        def add_row(r, carry):
            for j in range(dcat // _L):
                sl = pl.ds(j * _L, _L)
                ra_v[r, sl] = ra_v[r, sl] + rb_v[r, sl]
            return carry

        lax.fori_loop(0, _CH, add_row, 0)
        pltpu.sync_copy(ra_v, pre_hbm.at[pl.ds(base, _CH)])

    def body(i, carry):
        do_chunk(wid * cpw + i)
        return carry

    lax.fori_loop(0, cpw, body, 0)

    @pl.when(wid < extra)
    def _():
        do_chunk(_NW * cpw + wid)


def _edge_body(pre_ref, bond_ref, bg_ref, s32_ref, kb_ref, out_ref):
    pre = pre_ref[...]
    bond = bond_ref[...]
    row = bg_ref[0]
    ng = s32_ref.shape[0]
    onehot_t = (lax.broadcasted_iota(jnp.int32, (ng, row.shape[1]), 0)
                == row).astype(jnp.float32)
    contrib = lax.dot_general(onehot_t, s32_ref[...],
                              (((0,), (0,)), ((), ())),
                              preferred_element_type=jnp.float32)
    t = pre + jnp.dot(bond, kb_ref[...],
                      preferred_element_type=jnp.float32) + contrib
    d = out_ref.shape[1]
    ts = t[:, :d]
    tg = t[:, d:]
    sig = 1.0 / (1.0 + jnp.exp(-ts))
    out_ref[...] = sig * _softplus(tg)


_TR = 512
_TRP = _TR + 16

def _scatter_body(trans_hbm, src_hbm, agg_hbm, acc_sh, vbuf, ibuf, ibuf2, zbuf):
    e = src_hbm.shape[0]
    nt = acc_sh.shape[0]
    nch = e // _CH
    cpw = nch // _NW
    extra = nch - _NW * cpw
    da = acc_sh.shape[2]
    zr = zbuf.shape[0]
    cid = lax.axis_index("c")
    sid = lax.axis_index("s")
    wid = sid * _NC + cid

    def zero_row(r, carry):
        for j in range(da // _L):
            zbuf[r, pl.ds(j * _L, _L)] = jnp.zeros((_L,), jnp.float32)
        return carry

    lax.fori_loop(0, zr, zero_row, 0)
    for k in range(_TRP // zr):
        pltpu.sync_copy(zbuf, acc_sh.at[sid, pl.ds(k * zr, zr)])
        @pl.when(sid < nt - _NS)
        def _():
            pltpu.sync_copy(zbuf, acc_sh.at[_NS + sid, pl.ds(k * zr, zr)])
    plsc.subcore_barrier()

    def do_chunk(cidx):
        base = cidx * _CH
        pltpu.sync_copy(src_hbm.at[pl.ds(base, _CH)], ibuf)
        pltpu.sync_copy(trans_hbm.at[pl.ds(base, _CH)], vbuf)
        t_lo = ibuf[pl.ds(0, _L)][0] // _TR
        t_hi = ibuf[pl.ds(_CH - _L, _L)][_L - 1] // _TR

        def tbody(t, carry):
            shift = t * _TR - 8
            for j in range(_CH // _L):
                sl = pl.ds(j * _L, _L)
                ibuf2[sl] = jnp.clip(ibuf[sl] - shift, 0, _TR + 8)
            pltpu.sync_copy(vbuf, acc_sh.at[t].at[ibuf2], add=True)
            return carry

        lax.fori_loop(t_lo, t_hi + 1, tbody, 0)

    def body(i, carry):
        do_chunk(wid * cpw + i)
        return carry

    lax.fori_loop(0, cpw, body, 0)

    @pl.when(wid < extra)
    def _():
        do_chunk(_NW * cpw + wid)

    plsc.subcore_barrier()

    pltpu.sync_copy(acc_sh.at[sid, pl.ds(8, _TR)],
                    agg_hbm.at[pl.ds(cid * nt * _TR + sid * _TR, _TR)])
    @pl.when(sid < nt - _NS)
    def _():
        pltpu.sync_copy(acc_sh.at[_NS + sid, pl.ds(8, _TR)],
                        agg_hbm.at[pl.ds(cid * nt * _TR + (_NS + sid) * _TR, _TR)])


def _out_body(x_ref, a0_ref, a1_ref, out_ref):
    t = x_ref[...] + a0_ref[...] + a1_ref[...]
    out_ref[...] = _softplus(t)


def kernel(atom_features, bond_features, state_attrs, pair_indices,
           atom_graph_indices, bond_graph_indices,
           kernel_s, bias_s, kernel_g, bias_g):
    del atom_graph_indices
    n, da = atom_features.shape
    e, de = bond_features.shape
    ng, dst_dim = state_attrs.shape
    dcat = 2 * da

    kk = jnp.concatenate([kernel_s, kernel_g], axis=1)
    k_src = kk[:da]
    k_dst = kk[da:2 * da]
    k_state = kk[2 * da:2 * da + dst_dim]
    k_bond = kk[2 * da + dst_dim:]
    bias = jnp.concatenate([bias_s, bias_g]).reshape(1, dcat)

    src = pair_indices[:, 0]
    dst = pair_indices[:, 1]

    nb = 10
    bn = n // nb
    tsrc, tdst, s32 = pl.pallas_call(
        _proj_body,
        grid=(nb,),
        in_specs=[
            pl.BlockSpec((bn, da), lambda i: (i, 0)),
            pl.BlockSpec((da, dcat), lambda i: (0, 0)),
            pl.BlockSpec((da, dcat), lambda i: (0, 0)),
            pl.BlockSpec((ng, dst_dim), lambda i: (0, 0)),
            pl.BlockSpec((dst_dim, dcat), lambda i: (0, 0)),
            pl.BlockSpec((1, dcat), lambda i: (0, 0)),
        ],
        out_specs=[
            pl.BlockSpec((bn, dcat), lambda i: (i, 0)),
            pl.BlockSpec((bn, dcat), lambda i: (i, 0)),
            pl.BlockSpec((ng, dcat), lambda i: (0, 0)),
        ],
        out_shape=[
            jax.ShapeDtypeStruct((n, dcat), jnp.float32),
            jax.ShapeDtypeStruct((n, dcat), jnp.float32),
            jax.ShapeDtypeStruct((ng, dcat), jnp.float32),
        ],
    )(atom_features, k_src, k_dst, state_attrs, k_state, bias)

    mesh = plsc.VectorSubcoreMesh(core_axis_name="c", subcore_axis_name="s")
    pre = pl.kernel(
        _gather_body,
        mesh=mesh,
        out_type=jax.ShapeDtypeStruct((e, dcat), jnp.float32),
        scratch_types=[
            pltpu.VMEM((_CH,), jnp.int32),
            pltpu.VMEM((_CH,), jnp.int32),
            pltpu.VMEM((_CH, dcat), jnp.float32),
            pltpu.VMEM((_CH, dcat), jnp.float32),
            pltpu.SemaphoreType.DMA,
            pltpu.SemaphoreType.DMA,
        ],
    )(tsrc, tdst, src, dst)

    be = 512
    nbe = e // be
    bg3 = bond_graph_indices.reshape(nbe, 1, be)
    transformed = pl.pallas_call(
        _edge_body,
        grid=(nbe,),
        in_specs=[
            pl.BlockSpec((be, dcat), lambda i: (i, 0)),
            pl.BlockSpec((be, de), lambda i: (i, 0)),
            pl.BlockSpec((1, 1, be), lambda i: (i, 0, 0)),
            pl.BlockSpec((ng, dcat), lambda i: (0, 0)),
            pl.BlockSpec((de, dcat), lambda i: (0, 0)),
        ],
        out_specs=pl.BlockSpec((be, da), lambda i: (i, 0)),
        out_shape=jax.ShapeDtypeStruct((e, da), jnp.float32),
    )(pre, bond_features, bg3, s32, k_bond)

    nt = (n + _TR - 1) // _TR
    n_pad = nt * _TR
    agg = pl.kernel(
        _scatter_body,
        mesh=mesh,
        out_type=jax.ShapeDtypeStruct((_NC * n_pad, da), jnp.float32),
        scratch_types=[
            pltpu.VMEM_SHARED((nt, _TRP, da), jnp.float32),
            pltpu.VMEM((_CH, da), jnp.float32),
            pltpu.VMEM((_CH,), jnp.int32),
            pltpu.VMEM((_CH,), jnp.int32),
            pltpu.VMEM((_TRP // 3, da), jnp.float32),
        ],
    )(transformed, src)

    bn_e = 80
    nb_e = n // bn_e
    off = n_pad // bn_e
    out = pl.pallas_call(
        _out_body,
        grid=(nb_e,),
        in_specs=[
            pl.BlockSpec((bn_e, da), lambda i: (i, 0)),
            pl.BlockSpec((bn_e, da), lambda i: (i, 0)),
            pl.BlockSpec((bn_e, da), lambda i: (i + off, 0)),
        ],
        out_specs=pl.BlockSpec((bn_e, da), lambda i: (i, 0)),
        out_shape=jax.ShapeDtypeStruct((n, da), jnp.float32),
    )(atom_features, agg, agg)
    return out


# speedup vs baseline: 3.0587x; 1.1237x over previous
import functools

import jax
import jax.numpy as jnp
from jax import lax
from jax.experimental import pallas as pl
from jax.experimental.pallas import tpu as pltpu
from jax.experimental.pallas import tpu_sc as plsc

_NC = 2
_NS = 16
_NW = _NC * _NS
_L = 16
_CH = 128


def _softplus(t):
    return jnp.maximum(t, 0.0) + jnp.log(1.0 + jnp.exp(-jnp.abs(t)))


def _proj_body(x_ref, ks_ref, kd_ref, st_ref, kst_ref, b_ref,
               tsrc_ref, tdst_ref, s32_ref):
    x = x_ref[...]
    tsrc_ref[...] = jnp.dot(x, ks_ref[...], preferred_element_type=jnp.float32)
    tdst_ref[...] = jnp.dot(x, kd_ref[...], preferred_element_type=jnp.float32)

    @pl.when(pl.program_id(0) == 0)
    def _():
        s32_ref[...] = (
            jnp.dot(st_ref[...], kst_ref[...],
                    preferred_element_type=jnp.float32)
            + b_ref[...]
        )


_CHG = 64


def _gather_body(tsrc_hbm, tdst_hbm, src_hbm, dst_hbm, pre_hbm,
                 is0, is1, id0, id1, ra0, ra1, rb0, rb1,
                 si0, si1, sg0, sg1, sw0, sw1):
    e = src_hbm.shape[0]
    nch = e // _CHG
    cpw = nch // _NW
    extra = nch - _NW * cpw
    dcat = ra0.shape[1]
    wid = lax.axis_index("s") * _NC + lax.axis_index("c")
    isb, idb = (is0, is1), (id0, id1)
    rab, rbb = (ra0, ra1), (rb0, rb1)
    sib, sgb, swb = (si0, si1), (sg0, sg1), (sw0, sw1)

    def fire_idx(chunk, b):
        base = chunk * _CHG
        pltpu.async_copy(src_hbm.at[pl.ds(base, _CHG)], isb[b], sib[b])
        pltpu.async_copy(dst_hbm.at[pl.ds(base, _CHG)], idb[b], sib[b])

    def drain_idx(b):
        pltpu.make_async_copy(src_hbm.at[pl.ds(0, _CHG)], isb[b], sib[b]).wait()
        pltpu.make_async_copy(dst_hbm.at[pl.ds(0, _CHG)], idb[b], sib[b]).wait()

    def fire_gather(b):
        pltpu.async_copy(tsrc_hbm.at[isb[b]], rab[b], sgb[b])
        pltpu.async_copy(tdst_hbm.at[idb[b]], rbb[b], sgb[b])

    def drain_gather(b):
        pltpu.make_async_copy(tsrc_hbm.at[pl.ds(0, _CHG)], rab[b], sgb[b]).wait()
        pltpu.make_async_copy(tsrc_hbm.at[pl.ds(0, _CHG)], rbb[b], sgb[b]).wait()

    def drain_w(b):
        pltpu.make_async_copy(rab[b], pre_hbm.at[pl.ds(0, _CHG)], swb[b]).wait()

    def add_rows(b):
        def add_row(r, carry):
            for j in range(dcat // _L):
                sl = pl.ds(j * _L, _L)
                rab[b][r, sl] = rab[b][r, sl] + rbb[b][r, sl]
            return carry
        lax.fori_loop(0, _CHG, add_row, 0)

    first = wid * cpw
    fire_idx(first, 0)
    drain_idx(0)
    fire_gather(0)
    fire_idx(first + 1, 1)

    def step(i, b):
        @pl.when(i + 1 < cpw)
        def _():
            drain_idx(1 - b)
        @pl.when(i >= 1)
        def _():
            drain_w(1 - b)
        @pl.when(i + 1 < cpw)
        def _():
            fire_gather(1 - b)
        drain_gather(b)
        @pl.when(i + 2 < cpw)
        def _():
            fire_idx(first + i + 2, b)
        add_rows(b)
        pltpu.async_copy(rab[b], pre_hbm.at[pl.ds((first + i) * _CHG, _CHG)],
                         swb[b])

    def pair(p, carry):
        step(2 * p, 0)
        step(2 * p + 1, 1)
        return carry

    lax.fori_loop(0, cpw // 2, pair, 0)
    drain_w(1)

    @pl.when(wid < extra)
    def _():
        chunk = _NW * cpw + wid
        fire_idx(chunk, 0)
        drain_idx(0)
        fire_gather(0)
        drain_gather(0)
        add_rows(0)
        pltpu.async_copy(ra0, pre_hbm.at[pl.ds(chunk * _CHG, _CHG)], sw0)
        drain_w(0)


def _edge_body(pre_ref, bond_ref, bg_ref, s32_ref, kb_ref, out_ref):
    pre = pre_ref[...]
    bond = bond_ref[...]
    row = bg_ref[0]
    ng = s32_ref.shape[0]
    onehot_t = (lax.broadcasted_iota(jnp.int32, (ng, row.shape[1]), 0)
                == row).astype(jnp.float32)
    contrib = lax.dot_general(onehot_t, s32_ref[...],
                              (((0,), (0,)), ((), ())),
                              preferred_element_type=jnp.float32)
    t = pre + jnp.dot(bond, kb_ref[...],
                      preferred_element_type=jnp.float32) + contrib
    d = out_ref.shape[1]
    ts = t[:, :d]
    tg = t[:, d:]
    sig = 1.0 / (1.0 + jnp.exp(-ts))
    out_ref[...] = sig * _softplus(tg)


_TR = 512
_TRP = _TR + 16

def _scatter_body(trans_hbm, src_hbm, agg_hbm, acc_sh, vbuf, ibuf, ibuf2, zbuf):
    e = src_hbm.shape[0]
    nt = acc_sh.shape[0]
    nch = e // _CH
    cpw = nch // _NW
    extra = nch - _NW * cpw
    da = acc_sh.shape[2]
    zr = zbuf.shape[0]
    cid = lax.axis_index("c")
    sid = lax.axis_index("s")
    wid = sid * _NC + cid

    def zero_row(r, carry):
        for j in range(da // _L):
            zbuf[r, pl.ds(j * _L, _L)] = jnp.zeros((_L,), jnp.float32)
        return carry

    lax.fori_loop(0, zr, zero_row, 0)
    for k in range(_TRP // zr):
        pltpu.sync_copy(zbuf, acc_sh.at[sid, pl.ds(k * zr, zr)])
        @pl.when(sid < nt - _NS)
        def _():
            pltpu.sync_copy(zbuf, acc_sh.at[_NS + sid, pl.ds(k * zr, zr)])
    plsc.subcore_barrier()

    def do_chunk(cidx):
        base = cidx * _CH
        pltpu.sync_copy(src_hbm.at[pl.ds(base, _CH)], ibuf)
        pltpu.sync_copy(trans_hbm.at[pl.ds(base, _CH)], vbuf)
        t_lo = ibuf[pl.ds(0, _L)][0] // _TR
        t_hi = ibuf[pl.ds(_CH - _L, _L)][_L - 1] // _TR

        def tbody(t, carry):
            shift = t * _TR - 8
            for j in range(_CH // _L):
                sl = pl.ds(j * _L, _L)
                ibuf2[sl] = jnp.clip(ibuf[sl] - shift, 0, _TR + 8)
            pltpu.sync_copy(vbuf, acc_sh.at[t].at[ibuf2], add=True)
            return carry

        lax.fori_loop(t_lo, t_hi + 1, tbody, 0)

    def body(i, carry):
        do_chunk(wid * cpw + i)
        return carry

    lax.fori_loop(0, cpw, body, 0)

    @pl.when(wid < extra)
    def _():
        do_chunk(_NW * cpw + wid)

    plsc.subcore_barrier()

    pltpu.sync_copy(acc_sh.at[sid, pl.ds(8, _TR)],
                    agg_hbm.at[pl.ds(cid * nt * _TR + sid * _TR, _TR)])
    @pl.when(sid < nt - _NS)
    def _():
        pltpu.sync_copy(acc_sh.at[_NS + sid, pl.ds(8, _TR)],
                        agg_hbm.at[pl.ds(cid * nt * _TR + (_NS + sid) * _TR, _TR)])


def _out_body(x_ref, a0_ref, a1_ref, out_ref):
    t = x_ref[...] + a0_ref[...] + a1_ref[...]
    out_ref[...] = _softplus(t)


def kernel(atom_features, bond_features, state_attrs, pair_indices,
           atom_graph_indices, bond_graph_indices,
           kernel_s, bias_s, kernel_g, bias_g):
    del atom_graph_indices
    n, da = atom_features.shape
    e, de = bond_features.shape
    ng, dst_dim = state_attrs.shape
    dcat = 2 * da

    kk = jnp.concatenate([kernel_s, kernel_g], axis=1)
    k_src = kk[:da]
    k_dst = kk[da:2 * da]
    k_state = kk[2 * da:2 * da + dst_dim]
    k_bond = kk[2 * da + dst_dim:]
    bias = jnp.concatenate([bias_s, bias_g]).reshape(1, dcat)

    src = pair_indices[:, 0]
    dst = pair_indices[:, 1]

    nb = 10
    bn = n // nb
    tsrc, tdst, s32 = pl.pallas_call(
        _proj_body,
        grid=(nb,),
        in_specs=[
            pl.BlockSpec((bn, da), lambda i: (i, 0)),
            pl.BlockSpec((da, dcat), lambda i: (0, 0)),
            pl.BlockSpec((da, dcat), lambda i: (0, 0)),
            pl.BlockSpec((ng, dst_dim), lambda i: (0, 0)),
            pl.BlockSpec((dst_dim, dcat), lambda i: (0, 0)),
            pl.BlockSpec((1, dcat), lambda i: (0, 0)),
        ],
        out_specs=[
            pl.BlockSpec((bn, dcat), lambda i: (i, 0)),
            pl.BlockSpec((bn, dcat), lambda i: (i, 0)),
            pl.BlockSpec((ng, dcat), lambda i: (0, 0)),
        ],
        out_shape=[
            jax.ShapeDtypeStruct((n, dcat), jnp.float32),
            jax.ShapeDtypeStruct((n, dcat), jnp.float32),
            jax.ShapeDtypeStruct((ng, dcat), jnp.float32),
        ],
    )(atom_features, k_src, k_dst, state_attrs, k_state, bias)

    mesh = plsc.VectorSubcoreMesh(core_axis_name="c", subcore_axis_name="s")
    pre = pl.kernel(
        _gather_body,
        mesh=mesh,
        out_type=jax.ShapeDtypeStruct((e, dcat), jnp.float32),
        scratch_types=(
            [pltpu.VMEM((_CHG,), jnp.int32)] * 4
            + [pltpu.VMEM((_CHG, dcat), jnp.float32)] * 4
            + [pltpu.SemaphoreType.DMA] * 6
        ),
    )(tsrc, tdst, src, dst)

    be = 512
    nbe = e // be
    bg3 = bond_graph_indices.reshape(nbe, 1, be)
    transformed = pl.pallas_call(
        _edge_body,
        grid=(nbe,),
        in_specs=[
            pl.BlockSpec((be, dcat), lambda i: (i, 0)),
            pl.BlockSpec((be, de), lambda i: (i, 0)),
            pl.BlockSpec((1, 1, be), lambda i: (i, 0, 0)),
            pl.BlockSpec((ng, dcat), lambda i: (0, 0)),
            pl.BlockSpec((de, dcat), lambda i: (0, 0)),
        ],
        out_specs=pl.BlockSpec((be, da), lambda i: (i, 0)),
        out_shape=jax.ShapeDtypeStruct((e, da), jnp.float32),
    )(pre, bond_features, bg3, s32, k_bond)

    nt = (n + _TR - 1) // _TR
    n_pad = nt * _TR
    agg = pl.kernel(
        _scatter_body,
        mesh=mesh,
        out_type=jax.ShapeDtypeStruct((_NC * n_pad, da), jnp.float32),
        scratch_types=[
            pltpu.VMEM_SHARED((nt, _TRP, da), jnp.float32),
            pltpu.VMEM((_CH, da), jnp.float32),
            pltpu.VMEM((_CH,), jnp.int32),
            pltpu.VMEM((_CH,), jnp.int32),
            pltpu.VMEM((_TRP // 3, da), jnp.float32),
        ],
    )(transformed, src)

    bn_e = 80
    nb_e = n // bn_e
    off = n_pad // bn_e
    out = pl.pallas_call(
        _out_body,
        grid=(nb_e,),
        in_specs=[
            pl.BlockSpec((bn_e, da), lambda i: (i, 0)),
            pl.BlockSpec((bn_e, da), lambda i: (i, 0)),
            pl.BlockSpec((bn_e, da), lambda i: (i + off, 0)),
        ],
        out_specs=pl.BlockSpec((bn_e, da), lambda i: (i, 0)),
        out_shape=jax.ShapeDtypeStruct((n, da), jnp.float32),
    )(atom_features, agg, agg)
    return out


# speedup vs baseline: 3.6281x; 1.1862x over previous
import functools

import jax
import jax.numpy as jnp
from jax import lax
from jax.experimental import pallas as pl
from jax.experimental.pallas import tpu as pltpu
from jax.experimental.pallas import tpu_sc as plsc

_NC = 2
_NS = 16
_NW = _NC * _NS
_L = 16
_CH = 128


def _softplus(t):
    return jnp.maximum(t, 0.0) + jnp.log(1.0 + jnp.exp(-jnp.abs(t)))


def _round_bf16(a):
    u = lax.bitcast_convert_type(a, jnp.uint32)
    return (u + jnp.uint32(0x7FFF) + ((u >> 16) & jnp.uint32(1))) >> 16


def _pack_pair(a, b):
    return lax.bitcast_convert_type(
        (_round_bf16(b) << 16) | _round_bf16(a), jnp.int32)


def _proj_body(x_ref, ks_ref, kd_ref, st_ref, kst_ref, b_ref,
               tsrc_ref, tdst_ref, s32_ref):
    x = x_ref[...]
    d = x.shape[1]
    ms = jnp.dot(x, ks_ref[...], preferred_element_type=jnp.float32)
    md = jnp.dot(x, kd_ref[...], preferred_element_type=jnp.float32)
    tsrc_ref[...] = _pack_pair(ms[:, :d], ms[:, d:])
    tdst_ref[...] = _pack_pair(md[:, :d], md[:, d:])

    @pl.when(pl.program_id(0) == 0)
    def _():
        s32_ref[...] = (
            jnp.dot(st_ref[...], kst_ref[...],
                    preferred_element_type=jnp.float32)
            + b_ref[...]
        )


_CHG = 128


def _gather_body(tsrc_hbm, tdst_hbm, src_hbm, dst_hbm, ps_hbm, pd_hbm,
                 is0, is1, id0, id1, ra0, ra1, rb0, rb1,
                 si0, si1, sg0, sg1, sw0, sw1):
    e = src_hbm.shape[0]
    nch = e // _CHG
    cpw = nch // _NW
    extra = nch - _NW * cpw
    wid = lax.axis_index("s") * _NC + lax.axis_index("c")
    isb, idb = (is0, is1), (id0, id1)
    rab, rbb = (ra0, ra1), (rb0, rb1)
    sib, sgb, swb = (si0, si1), (sg0, sg1), (sw0, sw1)

    def fire_idx(chunk, b):
        base = chunk * _CHG
        pltpu.async_copy(src_hbm.at[pl.ds(base, _CHG)], isb[b], sib[b])
        pltpu.async_copy(dst_hbm.at[pl.ds(base, _CHG)], idb[b], sib[b])

    def drain_idx(b):
        pltpu.make_async_copy(src_hbm.at[pl.ds(0, _CHG)], isb[b], sib[b]).wait()
        pltpu.make_async_copy(dst_hbm.at[pl.ds(0, _CHG)], idb[b], sib[b]).wait()

    def fire_gather(b):
        pltpu.async_copy(tsrc_hbm.at[isb[b]], rab[b], sgb[b])
        pltpu.async_copy(tdst_hbm.at[idb[b]], rbb[b], sgb[b])

    def drain_gather(b):
        pltpu.make_async_copy(tsrc_hbm.at[pl.ds(0, _CHG)], rab[b], sgb[b]).wait()
        pltpu.make_async_copy(tsrc_hbm.at[pl.ds(0, _CHG)], rbb[b], sgb[b]).wait()

    def fire_w(chunk, b):
        pltpu.async_copy(rab[b], ps_hbm.at[pl.ds(chunk * _CHG, _CHG)], swb[b])
        pltpu.async_copy(rbb[b], pd_hbm.at[pl.ds(chunk * _CHG, _CHG)], swb[b])

    def drain_w(b):
        pltpu.make_async_copy(rab[b], ps_hbm.at[pl.ds(0, _CHG)], swb[b]).wait()
        pltpu.make_async_copy(rbb[b], pd_hbm.at[pl.ds(0, _CHG)], swb[b]).wait()

    first = wid * cpw
    fire_idx(first, 0)
    drain_idx(0)
    fire_gather(0)
    fire_idx(first + 1, 1)

    def step(i, b):
        @pl.when(i + 1 < cpw)
        def _():
            drain_idx(1 - b)
        @pl.when(i >= 1)
        def _():
            drain_w(1 - b)
        @pl.when(i + 1 < cpw)
        def _():
            fire_gather(1 - b)
        drain_gather(b)
        @pl.when(i + 2 < cpw)
        def _():
            fire_idx(first + i + 2, b)
        fire_w(first + i, b)

    def pair(p, carry):
        step(2 * p, 0)
        step(2 * p + 1, 1)
        return carry

    lax.fori_loop(0, cpw // 2, pair, 0)
    drain_w(1)

    @pl.when(wid < extra)
    def _():
        chunk = _NW * cpw + wid
        fire_idx(chunk, 0)
        drain_idx(0)
        fire_gather(0)
        drain_gather(0)
        fire_w(chunk, 0)
        drain_w(0)


def _unpack_pair(w):
    lo = lax.bitcast_convert_type(w << 16, jnp.float32)
    hi = lax.bitcast_convert_type(w & jnp.uint32(0xFFFF0000), jnp.float32)
    return lo, hi


def _edge_body(ps_ref, pd_ref, bond_ref, bg_ref, s32_ref, kb_ref, out_ref):
    ws = lax.bitcast_convert_type(ps_ref[...], jnp.uint32)
    wd = lax.bitcast_convert_type(pd_ref[...], jnp.uint32)
    ts_s, tg_s = _unpack_pair(ws)
    ts_d, tg_d = _unpack_pair(wd)
    ts_pre = ts_s + ts_d
    tg_pre = tg_s + tg_d
    bond = bond_ref[...]
    row = bg_ref[0]
    ng = s32_ref.shape[0]
    onehot_t = (lax.broadcasted_iota(jnp.int32, (ng, row.shape[1]), 0)
                == row).astype(jnp.float32)
    contrib = lax.dot_general(onehot_t, s32_ref[...],
                              (((0,), (0,)), ((), ())),
                              preferred_element_type=jnp.float32)
    bk = jnp.dot(bond, kb_ref[...], preferred_element_type=jnp.float32)
    d = out_ref.shape[1]
    ts = ts_pre + bk[:, :d] + contrib[:, :d]
    tg = tg_pre + bk[:, d:] + contrib[:, d:]
    sig = 1.0 / (1.0 + jnp.exp(-ts))
    out_ref[...] = sig * _softplus(tg)


_TR = 512
_TRP = _TR + 16

def _scatter_body(trans_hbm, src_hbm, agg_hbm, acc_sh, vbuf, ibuf, ibuf2, zbuf):
    e = src_hbm.shape[0]
    nt = acc_sh.shape[0]
    nch = e // _CH
    cpw = nch // _NW
    extra = nch - _NW * cpw
    da = acc_sh.shape[2]
    zr = zbuf.shape[0]
    cid = lax.axis_index("c")
    sid = lax.axis_index("s")
    wid = sid * _NC + cid

    def zero_row(r, carry):
        for j in range(da // _L):
            zbuf[r, pl.ds(j * _L, _L)] = jnp.zeros((_L,), jnp.float32)
        return carry

    lax.fori_loop(0, zr, zero_row, 0)
    for k in range(_TRP // zr):
        pltpu.sync_copy(zbuf, acc_sh.at[sid, pl.ds(k * zr, zr)])
        @pl.when(sid < nt - _NS)
        def _():
            pltpu.sync_copy(zbuf, acc_sh.at[_NS + sid, pl.ds(k * zr, zr)])
    plsc.subcore_barrier()

    def do_chunk(cidx):
        base = cidx * _CH
        pltpu.sync_copy(src_hbm.at[pl.ds(base, _CH)], ibuf)
        pltpu.sync_copy(trans_hbm.at[pl.ds(base, _CH)], vbuf)
        t_lo = ibuf[pl.ds(0, _L)][0] // _TR
        t_hi = ibuf[pl.ds(_CH - _L, _L)][_L - 1] // _TR

        def tbody(t, carry):
            shift = t * _TR - 8
            for j in range(_CH // _L):
                sl = pl.ds(j * _L, _L)
                ibuf2[sl] = jnp.clip(ibuf[sl] - shift, 0, _TR + 8)
            pltpu.sync_copy(vbuf, acc_sh.at[t].at[ibuf2], add=True)
            return carry

        lax.fori_loop(t_lo, t_hi + 1, tbody, 0)

    def body(i, carry):
        do_chunk(wid * cpw + i)
        return carry

    lax.fori_loop(0, cpw, body, 0)

    @pl.when(wid < extra)
    def _():
        do_chunk(_NW * cpw + wid)

    plsc.subcore_barrier()

    pltpu.sync_copy(acc_sh.at[sid, pl.ds(8, _TR)],
                    agg_hbm.at[pl.ds(cid * nt * _TR + sid * _TR, _TR)])
    @pl.when(sid < nt - _NS)
    def _():
        pltpu.sync_copy(acc_sh.at[_NS + sid, pl.ds(8, _TR)],
                        agg_hbm.at[pl.ds(cid * nt * _TR + (_NS + sid) * _TR, _TR)])


def _out_body(x_ref, a0_ref, a1_ref, out_ref):
    t = x_ref[...] + a0_ref[...] + a1_ref[...]
    out_ref[...] = _softplus(t)


def kernel(atom_features, bond_features, state_attrs, pair_indices,
           atom_graph_indices, bond_graph_indices,
           kernel_s, bias_s, kernel_g, bias_g):
    del atom_graph_indices
    n, da = atom_features.shape
    e, de = bond_features.shape
    ng, dst_dim = state_attrs.shape
    dcat = 2 * da

    kk = jnp.concatenate([kernel_s, kernel_g], axis=1)
    k_src = kk[:da]
    k_dst = kk[da:2 * da]
    k_state = kk[2 * da:2 * da + dst_dim]
    k_bond = kk[2 * da + dst_dim:]
    bias = jnp.concatenate([bias_s, bias_g]).reshape(1, dcat)

    src = pair_indices[:, 0]
    dst = pair_indices[:, 1]

    nb = 10
    bn = n // nb
    tsrc, tdst, s32 = pl.pallas_call(
        _proj_body,
        grid=(nb,),
        in_specs=[
            pl.BlockSpec((bn, da), lambda i: (i, 0)),
            pl.BlockSpec((da, dcat), lambda i: (0, 0)),
            pl.BlockSpec((da, dcat), lambda i: (0, 0)),
            pl.BlockSpec((ng, dst_dim), lambda i: (0, 0)),
            pl.BlockSpec((dst_dim, dcat), lambda i: (0, 0)),
            pl.BlockSpec((1, dcat), lambda i: (0, 0)),
        ],
        out_specs=[
            pl.BlockSpec((bn, da), lambda i: (i, 0)),
            pl.BlockSpec((bn, da), lambda i: (i, 0)),
            pl.BlockSpec((ng, dcat), lambda i: (0, 0)),
        ],
        out_shape=[
            jax.ShapeDtypeStruct((n, da), jnp.int32),
            jax.ShapeDtypeStruct((n, da), jnp.int32),
            jax.ShapeDtypeStruct((ng, dcat), jnp.float32),
        ],
    )(atom_features, k_src, k_dst, state_attrs, k_state, bias)

    mesh = plsc.VectorSubcoreMesh(core_axis_name="c", subcore_axis_name="s")
    pre_s, pre_d = pl.kernel(
        _gather_body,
        mesh=mesh,
        out_type=[
            jax.ShapeDtypeStruct((e, da), jnp.int32),
            jax.ShapeDtypeStruct((e, da), jnp.int32),
        ],
        scratch_types=(
            [pltpu.VMEM((_CHG,), jnp.int32)] * 4
            + [pltpu.VMEM((_CHG, da), jnp.int32)] * 4
            + [pltpu.SemaphoreType.DMA] * 6
        ),
    )(tsrc, tdst, src, dst)

    be = 512
    nbe = e // be
    bg3 = bond_graph_indices.reshape(nbe, 1, be)
    transformed = pl.pallas_call(
        _edge_body,
        grid=(nbe,),
        in_specs=[
            pl.BlockSpec((be, da), lambda i: (i, 0)),
            pl.BlockSpec((be, da), lambda i: (i, 0)),
            pl.BlockSpec((be, de), lambda i: (i, 0)),
            pl.BlockSpec((1, 1, be), lambda i: (i, 0, 0)),
            pl.BlockSpec((ng, dcat), lambda i: (0, 0)),
            pl.BlockSpec((de, dcat), lambda i: (0, 0)),
        ],
        out_specs=pl.BlockSpec((be, da), lambda i: (i, 0)),
        out_shape=jax.ShapeDtypeStruct((e, da), jnp.float32),
    )(pre_s, pre_d, bond_features, bg3, s32, k_bond)

    nt = (n + _TR - 1) // _TR
    n_pad = nt * _TR
    agg = pl.kernel(
        _scatter_body,
        mesh=mesh,
        out_type=jax.ShapeDtypeStruct((_NC * n_pad, da), jnp.float32),
        scratch_types=[
            pltpu.VMEM_SHARED((nt, _TRP, da), jnp.float32),
            pltpu.VMEM((_CH, da), jnp.float32),
            pltpu.VMEM((_CH,), jnp.int32),
            pltpu.VMEM((_CH,), jnp.int32),
            pltpu.VMEM((_TRP // 3, da), jnp.float32),
        ],
    )(transformed, src)

    bn_e = 80
    nb_e = n // bn_e
    off = n_pad // bn_e
    out = pl.pallas_call(
        _out_body,
        grid=(nb_e,),
        in_specs=[
            pl.BlockSpec((bn_e, da), lambda i: (i, 0)),
            pl.BlockSpec((bn_e, da), lambda i: (i, 0)),
            pl.BlockSpec((bn_e, da), lambda i: (i + off, 0)),
        ],
        out_specs=pl.BlockSpec((bn_e, da), lambda i: (i, 0)),
        out_shape=jax.ShapeDtypeStruct((n, da), jnp.float32),
    )(atom_features, agg, agg)
    return out


# speedup vs baseline: 3.8498x; 1.0611x over previous
import functools

import jax
import jax.numpy as jnp
from jax import lax
from jax.experimental import pallas as pl
from jax.experimental.pallas import tpu as pltpu
from jax.experimental.pallas import tpu_sc as plsc

_NC = 2
_NS = 16
_NW = _NC * _NS
_L = 16
_CH = 128


def _softplus(t):
    return jnp.maximum(t, 0.0) + jnp.log(1.0 + jnp.exp(-jnp.abs(t)))


def _round_bf16(a):
    u = lax.bitcast_convert_type(a, jnp.uint32)
    return (u + jnp.uint32(0x7FFF) + ((u >> 16) & jnp.uint32(1))) >> 16


def _pack_pair(a, b):
    return lax.bitcast_convert_type(
        (_round_bf16(b) << 16) | _round_bf16(a), jnp.int32)


def _proj_body(x_ref, ks_ref, kd_ref, st_ref, kst_ref, b_ref,
               tsrc_ref, tdst_ref, s32_ref):
    x = x_ref[...]
    d = x.shape[1]
    ms = jnp.dot(x, ks_ref[...], preferred_element_type=jnp.float32)
    md = jnp.dot(x, kd_ref[...], preferred_element_type=jnp.float32)
    tsrc_ref[...] = _pack_pair(ms[:, :d], ms[:, d:])
    tdst_ref[...] = _pack_pair(md[:, :d], md[:, d:])

    @pl.when(pl.program_id(0) == 0)
    def _():
        s32_ref[...] = (
            jnp.dot(st_ref[...], kst_ref[...],
                    preferred_element_type=jnp.float32)
            + b_ref[...]
        )


_CHG = 128


def _gather_body(tsrc_hbm, tdst_hbm, src_hbm, dst_hbm, ps_hbm, pd_hbm,
                 is0, is1, id0, id1, ra0, ra1, rb0, rb1,
                 si0, si1, sg0, sg1, sw0, sw1):
    e = src_hbm.shape[0]
    nch = e // _CHG
    cpw = nch // _NW
    extra = nch - _NW * cpw
    wid = lax.axis_index("s") * _NC + lax.axis_index("c")
    isb, idb = (is0, is1), (id0, id1)
    rab, rbb = (ra0, ra1), (rb0, rb1)
    sib, sgb, swb = (si0, si1), (sg0, sg1), (sw0, sw1)

    def fire_idx(chunk, b):
        base = chunk * _CHG
        pltpu.async_copy(src_hbm.at[pl.ds(base, _CHG)], isb[b], sib[b])
        pltpu.async_copy(dst_hbm.at[pl.ds(base, _CHG)], idb[b], sib[b])

    def drain_idx(b):
        pltpu.make_async_copy(src_hbm.at[pl.ds(0, _CHG)], isb[b], sib[b]).wait()
        pltpu.make_async_copy(dst_hbm.at[pl.ds(0, _CHG)], idb[b], sib[b]).wait()

    def fire_gather(b):
        pltpu.async_copy(tsrc_hbm.at[isb[b]], rab[b], sgb[b])
        pltpu.async_copy(tdst_hbm.at[idb[b]], rbb[b], sgb[b])

    def drain_gather(b):
        pltpu.make_async_copy(tsrc_hbm.at[pl.ds(0, _CHG)], rab[b], sgb[b]).wait()
        pltpu.make_async_copy(tsrc_hbm.at[pl.ds(0, _CHG)], rbb[b], sgb[b]).wait()

    def fire_w(chunk, b):
        pltpu.async_copy(rab[b], ps_hbm.at[pl.ds(chunk * _CHG, _CHG)], swb[b])
        pltpu.async_copy(rbb[b], pd_hbm.at[pl.ds(chunk * _CHG, _CHG)], swb[b])

    def drain_w(b):
        pltpu.make_async_copy(rab[b], ps_hbm.at[pl.ds(0, _CHG)], swb[b]).wait()
        pltpu.make_async_copy(rbb[b], pd_hbm.at[pl.ds(0, _CHG)], swb[b]).wait()

    first = wid * cpw
    fire_idx(first, 0)
    drain_idx(0)
    fire_gather(0)
    fire_idx(first + 1, 1)

    def step(i, b):
        @pl.when(i + 1 < cpw)
        def _():
            drain_idx(1 - b)
        @pl.when(i >= 1)
        def _():
            drain_w(1 - b)
        @pl.when(i + 1 < cpw)
        def _():
            fire_gather(1 - b)
        drain_gather(b)
        @pl.when(i + 2 < cpw)
        def _():
            fire_idx(first + i + 2, b)
        fire_w(first + i, b)

    def pair(p, carry):
        step(2 * p, 0)
        step(2 * p + 1, 1)
        return carry

    lax.fori_loop(0, cpw // 2, pair, 0)
    drain_w(1)

    @pl.when(wid < extra)
    def _():
        chunk = _NW * cpw + wid
        fire_idx(chunk, 0)
        drain_idx(0)
        fire_gather(0)
        drain_gather(0)
        fire_w(chunk, 0)
        drain_w(0)


def _unpack_pair(w):
    lo = lax.bitcast_convert_type(w << 16, jnp.float32)
    hi = lax.bitcast_convert_type(w & jnp.uint32(0xFFFF0000), jnp.float32)
    return lo, hi


def _edge_body(ps_ref, pd_ref, bond_ref, bg_ref, s32_ref, kb_ref, out_ref):
    ws = lax.bitcast_convert_type(ps_ref[...], jnp.uint32)
    wd = lax.bitcast_convert_type(pd_ref[...], jnp.uint32)
    ts_s, tg_s = _unpack_pair(ws)
    ts_d, tg_d = _unpack_pair(wd)
    ts_pre = ts_s + ts_d
    tg_pre = tg_s + tg_d
    bond = bond_ref[...]
    row = bg_ref[0]
    ng = s32_ref.shape[0]
    onehot_t = (lax.broadcasted_iota(jnp.int32, (ng, row.shape[1]), 0)
                == row).astype(jnp.float32)
    contrib = lax.dot_general(onehot_t, s32_ref[...],
                              (((0,), (0,)), ((), ())),
                              preferred_element_type=jnp.float32)
    bk = jnp.dot(bond, kb_ref[...], preferred_element_type=jnp.float32)
    d = out_ref.shape[1]
    ts = ts_pre + bk[:, :d] + contrib[:, :d]
    tg = tg_pre + bk[:, d:] + contrib[:, d:]
    sig = 1.0 / (1.0 + jnp.exp(-ts))
    out_ref[...] = sig * _softplus(tg)


_TR = 512
_TRP = _TR + 16

_CHD = 64

def _scatter_body(trans_hbm, src_hbm, agg_hbm, acc_sh, vb0, vb1, ib0, ib1,
                  ibuf2, zbuf, sp0, sp1):
    e = src_hbm.shape[0]
    nt = acc_sh.shape[0]
    nch = e // _CHD
    cpw = nch // _NW
    extra = nch - _NW * cpw
    da = acc_sh.shape[2]
    zr = zbuf.shape[0]
    cid = lax.axis_index("c")
    sid = lax.axis_index("s")
    wid = sid * _NC + cid
    vbb, ibb, spb = (vb0, vb1), (ib0, ib1), (sp0, sp1)

    def zero_row(r, carry):
        for j in range(da // _L):
            zbuf[r, pl.ds(j * _L, _L)] = jnp.zeros((_L,), jnp.float32)
        return carry

    lax.fori_loop(0, zr, zero_row, 0)
    for k in range(_TRP // zr):
        pltpu.sync_copy(zbuf, acc_sh.at[sid, pl.ds(k * zr, zr)])
        @pl.when(sid < nt - _NS)
        def _():
            pltpu.sync_copy(zbuf, acc_sh.at[_NS + sid, pl.ds(k * zr, zr)])
    plsc.subcore_barrier()

    def fire_p(cidx, b):
        base = cidx * _CHD
        pltpu.async_copy(src_hbm.at[pl.ds(base, _CHD)], ibb[b], spb[b])
        pltpu.async_copy(trans_hbm.at[pl.ds(base, _CHD)], vbb[b], spb[b])

    def drain_p(b):
        pltpu.make_async_copy(src_hbm.at[pl.ds(0, _CHD)], ibb[b], spb[b]).wait()
        pltpu.make_async_copy(trans_hbm.at[pl.ds(0, _CHD)], vbb[b],
                              spb[b]).wait()

    def scatter_chunk(b):
        ibuf = ibb[b]
        t_lo = ibuf[pl.ds(0, _L)][0] // _TR
        t_hi = ibuf[pl.ds(_CHD - _L, _L)][_L - 1] // _TR

        def tbody(t, carry):
            shift = t * _TR - 8
            for j in range(_CHD // _L):
                sl = pl.ds(j * _L, _L)
                ibuf2[sl] = jnp.clip(ibuf[sl] - shift, 0, _TR + 8)
            pltpu.sync_copy(vbb[b], acc_sh.at[t].at[ibuf2], add=True)
            return carry

        lax.fori_loop(t_lo, t_hi + 1, tbody, 0)

    first = wid * cpw
    fire_p(first, 0)
    fire_p(first + 1, 1)

    def step(i, b):
        drain_p(b)
        scatter_chunk(b)
        @pl.when(i + 2 < cpw)
        def _():
            fire_p(first + i + 2, b)

    def pair(p, carry):
        step(2 * p, 0)
        step(2 * p + 1, 1)
        return carry

    lax.fori_loop(0, cpw // 2, pair, 0)

    @pl.when(wid < extra)
    def _():
        chunk = _NW * cpw + wid
        fire_p(chunk, 0)
        drain_p(0)
        scatter_chunk(0)

    plsc.subcore_barrier()

    pltpu.sync_copy(acc_sh.at[sid, pl.ds(8, _TR)],
                    agg_hbm.at[pl.ds(cid * nt * _TR + sid * _TR, _TR)])
    @pl.when(sid < nt - _NS)
    def _():
        pltpu.sync_copy(acc_sh.at[_NS + sid, pl.ds(8, _TR)],
                        agg_hbm.at[pl.ds(cid * nt * _TR + (_NS + sid) * _TR, _TR)])


def _out_body(x_ref, a0_ref, a1_ref, out_ref):
    t = x_ref[...] + a0_ref[...] + a1_ref[...]
    out_ref[...] = _softplus(t)


def kernel(atom_features, bond_features, state_attrs, pair_indices,
           atom_graph_indices, bond_graph_indices,
           kernel_s, bias_s, kernel_g, bias_g):
    del atom_graph_indices
    n, da = atom_features.shape
    e, de = bond_features.shape
    ng, dst_dim = state_attrs.shape
    dcat = 2 * da

    kk = jnp.concatenate([kernel_s, kernel_g], axis=1)
    k_src = kk[:da]
    k_dst = kk[da:2 * da]
    k_state = kk[2 * da:2 * da + dst_dim]
    k_bond = kk[2 * da + dst_dim:]
    bias = jnp.concatenate([bias_s, bias_g]).reshape(1, dcat)

    src = pair_indices[:, 0]
    dst = pair_indices[:, 1]

    nb = 10
    bn = n // nb
    tsrc, tdst, s32 = pl.pallas_call(
        _proj_body,
        grid=(nb,),
        in_specs=[
            pl.BlockSpec((bn, da), lambda i: (i, 0)),
            pl.BlockSpec((da, dcat), lambda i: (0, 0)),
            pl.BlockSpec((da, dcat), lambda i: (0, 0)),
            pl.BlockSpec((ng, dst_dim), lambda i: (0, 0)),
            pl.BlockSpec((dst_dim, dcat), lambda i: (0, 0)),
            pl.BlockSpec((1, dcat), lambda i: (0, 0)),
        ],
        out_specs=[
            pl.BlockSpec((bn, da), lambda i: (i, 0)),
            pl.BlockSpec((bn, da), lambda i: (i, 0)),
            pl.BlockSpec((ng, dcat), lambda i: (0, 0)),
        ],
        out_shape=[
            jax.ShapeDtypeStruct((n, da), jnp.int32),
            jax.ShapeDtypeStruct((n, da), jnp.int32),
            jax.ShapeDtypeStruct((ng, dcat), jnp.float32),
        ],
    )(atom_features, k_src, k_dst, state_attrs, k_state, bias)

    mesh = plsc.VectorSubcoreMesh(core_axis_name="c", subcore_axis_name="s")
    pre_s, pre_d = pl.kernel(
        _gather_body,
        mesh=mesh,
        out_type=[
            jax.ShapeDtypeStruct((e, da), jnp.int32),
            jax.ShapeDtypeStruct((e, da), jnp.int32),
        ],
        scratch_types=(
            [pltpu.VMEM((_CHG,), jnp.int32)] * 4
            + [pltpu.VMEM((_CHG, da), jnp.int32)] * 4
            + [pltpu.SemaphoreType.DMA] * 6
        ),
    )(tsrc, tdst, src, dst)

    be = 512
    nbe = e // be
    bg3 = bond_graph_indices.reshape(nbe, 1, be)
    transformed = pl.pallas_call(
        _edge_body,
        grid=(nbe,),
        in_specs=[
            pl.BlockSpec((be, da), lambda i: (i, 0)),
            pl.BlockSpec((be, da), lambda i: (i, 0)),
            pl.BlockSpec((be, de), lambda i: (i, 0)),
            pl.BlockSpec((1, 1, be), lambda i: (i, 0, 0)),
            pl.BlockSpec((ng, dcat), lambda i: (0, 0)),
            pl.BlockSpec((de, dcat), lambda i: (0, 0)),
        ],
        out_specs=pl.BlockSpec((be, da), lambda i: (i, 0)),
        out_shape=jax.ShapeDtypeStruct((e, da), jnp.float32),
    )(pre_s, pre_d, bond_features, bg3, s32, k_bond)

    nt = (n + _TR - 1) // _TR
    n_pad = nt * _TR
    agg = pl.kernel(
        _scatter_body,
        mesh=mesh,
        out_type=jax.ShapeDtypeStruct((_NC * n_pad, da), jnp.float32),
        scratch_types=[
            pltpu.VMEM_SHARED((nt, _TRP, da), jnp.float32),
            pltpu.VMEM((_CHD, da), jnp.float32),
            pltpu.VMEM((_CHD, da), jnp.float32),
            pltpu.VMEM((_CHD,), jnp.int32),
            pltpu.VMEM((_CHD,), jnp.int32),
            pltpu.VMEM((_CHD,), jnp.int32),
            pltpu.VMEM((_TRP // 6, da), jnp.float32),
            pltpu.SemaphoreType.DMA,
            pltpu.SemaphoreType.DMA,
        ],
    )(transformed, src)

    bn_e = 80
    nb_e = n // bn_e
    off = n_pad // bn_e
    out = pl.pallas_call(
        _out_body,
        grid=(nb_e,),
        in_specs=[
            pl.BlockSpec((bn_e, da), lambda i: (i, 0)),
            pl.BlockSpec((bn_e, da), lambda i: (i, 0)),
            pl.BlockSpec((bn_e, da), lambda i: (i + off, 0)),
        ],
        out_specs=pl.BlockSpec((bn_e, da), lambda i: (i, 0)),
        out_shape=jax.ShapeDtypeStruct((n, da), jnp.float32),
    )(atom_features, agg, agg)
    return out


# speedup vs baseline: 4.2591x; 1.1063x over previous
import functools

import jax
import jax.numpy as jnp
from jax import lax
from jax.experimental import pallas as pl
from jax.experimental.pallas import tpu as pltpu
from jax.experimental.pallas import tpu_sc as plsc

_NC = 2
_NS = 16
_NW = _NC * _NS
_L = 16
_CH = 128


def _softplus(t):
    return jnp.maximum(t, 0.0) + jnp.log(1.0 + jnp.exp(-jnp.abs(t)))


def _round_bf16(a):
    u = lax.bitcast_convert_type(a, jnp.uint32)
    return (u + jnp.uint32(0x7FFF) + ((u >> 16) & jnp.uint32(1))) >> 16


def _pack_pair(a, b):
    return lax.bitcast_convert_type(
        (_round_bf16(b) << 16) | _round_bf16(a), jnp.int32)


def _proj_body(x_ref, ks_ref, kd_ref, st_ref, kst_ref, b_ref,
               tsrc_ref, tdst_ref, s32_ref):
    x = x_ref[...]
    d = x.shape[1]
    ms = jnp.dot(x, ks_ref[...], preferred_element_type=jnp.float32)
    md = jnp.dot(x, kd_ref[...], preferred_element_type=jnp.float32)
    tsrc_ref[...] = _pack_pair(ms[:, :d], ms[:, d:])
    tdst_ref[...] = _pack_pair(md[:, :d], md[:, d:])

    @pl.when(pl.program_id(0) == 0)
    def _():
        s32_ref[...] = (
            jnp.dot(st_ref[...], kst_ref[...],
                    preferred_element_type=jnp.float32)
            + b_ref[...]
        )


_CHG = 128


def _gather_body(tsrc_hbm, tdst_hbm, src_hbm, dst_hbm, ps_hbm, pd_hbm,
                 is0, is1, id0, id1, ra0, ra1, rb0, rb1,
                 si0, si1, sg0, sg1, sw0, sw1):
    e = src_hbm.shape[0]
    nch = e // _CHG
    cpw = nch // _NW
    extra = nch - _NW * cpw
    wid = lax.axis_index("s") * _NC + lax.axis_index("c")
    isb, idb = (is0, is1), (id0, id1)
    rab, rbb = (ra0, ra1), (rb0, rb1)
    sib, sgb, swb = (si0, si1), (sg0, sg1), (sw0, sw1)

    def fire_idx(chunk, b):
        base = chunk * _CHG
        pltpu.async_copy(src_hbm.at[pl.ds(base, _CHG)], isb[b], sib[b])
        pltpu.async_copy(dst_hbm.at[pl.ds(base, _CHG)], idb[b], sib[b])

    def drain_idx(b):
        pltpu.make_async_copy(src_hbm.at[pl.ds(0, _CHG)], isb[b], sib[b]).wait()
        pltpu.make_async_copy(dst_hbm.at[pl.ds(0, _CHG)], idb[b], sib[b]).wait()

    def fire_gather(b):
        pltpu.async_copy(tsrc_hbm.at[isb[b]], rab[b], sgb[b])
        pltpu.async_copy(tdst_hbm.at[idb[b]], rbb[b], sgb[b])

    def drain_gather(b):
        pltpu.make_async_copy(tsrc_hbm.at[pl.ds(0, _CHG)], rab[b], sgb[b]).wait()
        pltpu.make_async_copy(tsrc_hbm.at[pl.ds(0, _CHG)], rbb[b], sgb[b]).wait()

    def fire_w(chunk, b):
        pltpu.async_copy(rab[b], ps_hbm.at[pl.ds(chunk * _CHG, _CHG)], swb[b])
        pltpu.async_copy(rbb[b], pd_hbm.at[pl.ds(chunk * _CHG, _CHG)], swb[b])

    def drain_w(b):
        pltpu.make_async_copy(rab[b], ps_hbm.at[pl.ds(0, _CHG)], swb[b]).wait()
        pltpu.make_async_copy(rbb[b], pd_hbm.at[pl.ds(0, _CHG)], swb[b]).wait()

    first = wid * cpw
    fire_idx(first, 0)
    drain_idx(0)
    fire_gather(0)
    fire_idx(first + 1, 1)

    def guard(cond, fn):
        if isinstance(cond, bool):
            if cond:
                fn()
        else:
            pl.when(cond)(fn)

    def step(i, b):
        guard(i + 1 < cpw, lambda: drain_idx(1 - b))
        guard(i >= 1, lambda: drain_w(1 - b))
        guard(i + 1 < cpw, lambda: fire_gather(1 - b))
        drain_gather(b)
        guard(i + 2 < cpw, lambda: fire_idx(first + i + 2, b))
        fire_w(first + i, b)

    def pair(p, carry):
        step(2 * p, 0)
        step(2 * p + 1, 1)
        return carry

    lax.fori_loop(0, cpw // 2, pair, 0)
    if cpw % 2:
        step(cpw - 1, 0)
    drain_w((cpw - 1) % 2)

    @pl.when(wid < extra)
    def _():
        chunk = _NW * cpw + wid
        fire_idx(chunk, 0)
        drain_idx(0)
        fire_gather(0)
        drain_gather(0)
        fire_w(chunk, 0)
        drain_w(0)


def _unpack_pair(w):
    lo = lax.bitcast_convert_type(w << 16, jnp.float32)
    hi = lax.bitcast_convert_type(w & jnp.uint32(0xFFFF0000), jnp.float32)
    return lo, hi


def _edge_body(ps_ref, pd_ref, bond_ref, bg_ref, s32_ref, kb_ref, out_ref):
    ws = lax.bitcast_convert_type(ps_ref[...], jnp.uint32)
    wd = lax.bitcast_convert_type(pd_ref[...], jnp.uint32)
    ts_s, tg_s = _unpack_pair(ws)
    ts_d, tg_d = _unpack_pair(wd)
    ts_pre = ts_s + ts_d
    tg_pre = tg_s + tg_d
    bond = bond_ref[...]
    row = bg_ref[0]
    ng = s32_ref.shape[0]
    onehot_t = (lax.broadcasted_iota(jnp.int32, (ng, row.shape[1]), 0)
                == row).astype(jnp.float32)
    contrib = lax.dot_general(onehot_t, s32_ref[...],
                              (((0,), (0,)), ((), ())),
                              preferred_element_type=jnp.float32)
    bk = jnp.dot(bond, kb_ref[...], preferred_element_type=jnp.float32)
    d = out_ref.shape[1]
    ts = ts_pre + bk[:, :d] + contrib[:, :d]
    tg = tg_pre + bk[:, d:] + contrib[:, d:]
    sig = 1.0 / (1.0 + jnp.exp(-ts))
    out_ref[...] = sig * _softplus(tg)


_TR = 512
_TRP = _TR + 16

_CHD = 64

def _scatter_body(tr0_hbm, tr1_hbm, src_hbm, agg_hbm, acc_sh, vb0, vb1,
                  ib0, ib1, ibuf2, zbuf, sp0, sp1):
    e = src_hbm.shape[0]
    nt = acc_sh.shape[0]
    nch = e // _CHD
    cpw = nch // _NW
    extra = nch - _NW * cpw
    da = acc_sh.shape[2]
    zr = zbuf.shape[0]
    cid = lax.axis_index("c")
    sid = lax.axis_index("s")
    wid = sid * _NC + cid
    vbb, ibb, spb = (vb0, vb1), (ib0, ib1), (sp0, sp1)

    def zero_row(r, carry):
        for j in range(da // _L):
            zbuf[r, pl.ds(j * _L, _L)] = jnp.zeros((_L,), jnp.float32)
        return carry

    lax.fori_loop(0, zr, zero_row, 0)
    for k in range(_TRP // zr):
        pltpu.sync_copy(zbuf, acc_sh.at[sid, pl.ds(k * zr, zr)])
        @pl.when(sid < nt - _NS)
        def _():
            pltpu.sync_copy(zbuf, acc_sh.at[_NS + sid, pl.ds(k * zr, zr)])
    plsc.subcore_barrier()

    nch0 = nch // 2

    def fire_p(cidx, b):
        base = cidx * _CHD
        pltpu.async_copy(src_hbm.at[pl.ds(base, _CHD)], ibb[b], spb[b])
        @pl.when(cidx < nch0)
        def _():
            pltpu.async_copy(tr0_hbm.at[pl.ds(base, _CHD)], vbb[b], spb[b])
        @pl.when(cidx >= nch0)
        def _():
            pltpu.async_copy(tr1_hbm.at[pl.ds(base - nch0 * _CHD, _CHD)],
                             vbb[b], spb[b])

    def drain_p(b):
        pltpu.make_async_copy(src_hbm.at[pl.ds(0, _CHD)], ibb[b], spb[b]).wait()
        pltpu.make_async_copy(tr0_hbm.at[pl.ds(0, _CHD)], vbb[b],
                              spb[b]).wait()

    def scatter_chunk(b):
        ibuf = ibb[b]
        t_lo = ibuf[pl.ds(0, _L)][0] // _TR
        t_hi = ibuf[pl.ds(_CHD - _L, _L)][_L - 1] // _TR

        def tbody(t, carry):
            shift = t * _TR - 8
            for j in range(_CHD // _L):
                sl = pl.ds(j * _L, _L)
                ibuf2[sl] = jnp.clip(ibuf[sl] - shift, 0, _TR + 8)
            pltpu.sync_copy(vbb[b], acc_sh.at[t].at[ibuf2], add=True)
            return carry

        lax.fori_loop(t_lo, t_hi + 1, tbody, 0)

    first = wid * cpw
    fire_p(first, 0)
    fire_p(first + 1, 1)

    def step(i, b):
        drain_p(b)
        scatter_chunk(b)
        @pl.when(i + 2 < cpw)
        def _():
            fire_p(first + i + 2, b)

    def pair(p, carry):
        step(2 * p, 0)
        step(2 * p + 1, 1)
        return carry

    lax.fori_loop(0, cpw // 2, pair, 0)

    @pl.when(wid < extra)
    def _():
        chunk = _NW * cpw + wid
        fire_p(chunk, 0)
        drain_p(0)
        scatter_chunk(0)

    plsc.subcore_barrier()

    pltpu.sync_copy(acc_sh.at[sid, pl.ds(8, _TR)],
                    agg_hbm.at[pl.ds(cid * nt * _TR + sid * _TR, _TR)])
    @pl.when(sid < nt - _NS)
    def _():
        pltpu.sync_copy(acc_sh.at[_NS + sid, pl.ds(8, _TR)],
                        agg_hbm.at[pl.ds(cid * nt * _TR + (_NS + sid) * _TR, _TR)])


def _out_body(x_ref, a0_ref, a1_ref, out_ref):
    t = x_ref[...] + a0_ref[...] + a1_ref[...]
    out_ref[...] = _softplus(t)


def kernel(atom_features, bond_features, state_attrs, pair_indices,
           atom_graph_indices, bond_graph_indices,
           kernel_s, bias_s, kernel_g, bias_g):
    del atom_graph_indices
    n, da = atom_features.shape
    e, de = bond_features.shape
    ng, dst_dim = state_attrs.shape
    dcat = 2 * da

    kk = jnp.concatenate([kernel_s, kernel_g], axis=1)
    k_src = kk[:da]
    k_dst = kk[da:2 * da]
    k_state = kk[2 * da:2 * da + dst_dim]
    k_bond = kk[2 * da + dst_dim:]
    bias = jnp.concatenate([bias_s, bias_g]).reshape(1, dcat)

    src = pair_indices[:, 0]
    dst = pair_indices[:, 1]

    nb = 10
    bn = n // nb
    tsrc, tdst, s32 = pl.pallas_call(
        _proj_body,
        grid=(nb,),
        in_specs=[
            pl.BlockSpec((bn, da), lambda i: (i, 0)),
            pl.BlockSpec((da, dcat), lambda i: (0, 0)),
            pl.BlockSpec((da, dcat), lambda i: (0, 0)),
            pl.BlockSpec((ng, dst_dim), lambda i: (0, 0)),
            pl.BlockSpec((dst_dim, dcat), lambda i: (0, 0)),
            pl.BlockSpec((1, dcat), lambda i: (0, 0)),
        ],
        out_specs=[
            pl.BlockSpec((bn, da), lambda i: (i, 0)),
            pl.BlockSpec((bn, da), lambda i: (i, 0)),
            pl.BlockSpec((ng, dcat), lambda i: (0, 0)),
        ],
        out_shape=[
            jax.ShapeDtypeStruct((n, da), jnp.int32),
            jax.ShapeDtypeStruct((n, da), jnp.int32),
            jax.ShapeDtypeStruct((ng, dcat), jnp.float32),
        ],
    )(atom_features, k_src, k_dst, state_attrs, k_state, bias)

    mesh = plsc.VectorSubcoreMesh(core_axis_name="c", subcore_axis_name="s")
    eh = e // 2
    be = 640
    nbe_h = eh // be
    bg3 = bond_graph_indices.reshape(e // be, 1, be)
    pre_halves = []
    for h in range(2):
        pre_s, pre_d = pl.kernel(
            _gather_body,
            mesh=mesh,
            out_type=[
                jax.ShapeDtypeStruct((eh, da), jnp.int32),
                jax.ShapeDtypeStruct((eh, da), jnp.int32),
            ],
            scratch_types=(
                [pltpu.VMEM((_CHG,), jnp.int32)] * 4
                + [pltpu.VMEM((_CHG, da), jnp.int32)] * 4
                + [pltpu.SemaphoreType.DMA] * 6
            ),
        )(tsrc, tdst, src[h * eh:(h + 1) * eh], dst[h * eh:(h + 1) * eh])
        pre_halves.append((pre_s, pre_d))

    trans_halves = []
    for h in range(2):
        off = h * nbe_h
        trans_halves.append(pl.pallas_call(
            _edge_body,
            grid=(nbe_h,),
            in_specs=[
                pl.BlockSpec((be, da), lambda i: (i, 0)),
                pl.BlockSpec((be, da), lambda i: (i, 0)),
                pl.BlockSpec((be, de), lambda i, off=off: (i + off, 0)),
                pl.BlockSpec((1, 1, be), lambda i, off=off: (i + off, 0, 0)),
                pl.BlockSpec((ng, dcat), lambda i: (0, 0)),
                pl.BlockSpec((de, dcat), lambda i: (0, 0)),
            ],
            out_specs=pl.BlockSpec((be, da), lambda i: (i, 0)),
            out_shape=jax.ShapeDtypeStruct((eh, da), jnp.float32),
        )(pre_halves[h][0], pre_halves[h][1], bond_features, bg3, s32, k_bond))

    nt = (n + _TR - 1) // _TR
    n_pad = nt * _TR
    agg = pl.kernel(
        _scatter_body,
        mesh=mesh,
        out_type=jax.ShapeDtypeStruct((_NC * n_pad, da), jnp.float32),
        scratch_types=[
            pltpu.VMEM_SHARED((nt, _TRP, da), jnp.float32),
            pltpu.VMEM((_CHD, da), jnp.float32),
            pltpu.VMEM((_CHD, da), jnp.float32),
            pltpu.VMEM((_CHD,), jnp.int32),
            pltpu.VMEM((_CHD,), jnp.int32),
            pltpu.VMEM((_CHD,), jnp.int32),
            pltpu.VMEM((_TRP // 6, da), jnp.float32),
            pltpu.SemaphoreType.DMA,
            pltpu.SemaphoreType.DMA,
        ],
    )(trans_halves[0], trans_halves[1], src)

    bn_e = 80
    nb_e = n // bn_e
    off = n_pad // bn_e
    out = pl.pallas_call(
        _out_body,
        grid=(nb_e,),
        in_specs=[
            pl.BlockSpec((bn_e, da), lambda i: (i, 0)),
            pl.BlockSpec((bn_e, da), lambda i: (i, 0)),
            pl.BlockSpec((bn_e, da), lambda i: (i + off, 0)),
        ],
        out_specs=pl.BlockSpec((bn_e, da), lambda i: (i, 0)),
        out_shape=jax.ShapeDtypeStruct((n, da), jnp.float32),
    )(atom_features, agg, agg)
    return out


# speedup vs baseline: 4.4338x; 1.0410x over previous
import functools

import jax
import jax.numpy as jnp
from jax import lax
from jax.experimental import pallas as pl
from jax.experimental.pallas import tpu as pltpu
from jax.experimental.pallas import tpu_sc as plsc

_NC = 2
_NS = 16
_NW = _NC * _NS
_L = 16
_CH = 128


def _softplus(t):
    return jnp.maximum(t, 0.0) + jnp.log(1.0 + jnp.exp(-jnp.abs(t)))


def _round_bf16(a):
    u = lax.bitcast_convert_type(a, jnp.uint32)
    return (u + jnp.uint32(0x7FFF) + ((u >> 16) & jnp.uint32(1))) >> 16


def _pack_pair(a, b):
    return lax.bitcast_convert_type(
        (_round_bf16(b) << 16) | _round_bf16(a), jnp.int32)


def _proj_body(x_ref, ks_ref, kd_ref, st_ref, kst_ref, b_ref,
               tsrc_ref, tdst_ref, s32_ref):
    x = x_ref[...]
    d = x.shape[1]
    ms = jnp.dot(x, ks_ref[...], preferred_element_type=jnp.float32)
    md = jnp.dot(x, kd_ref[...], preferred_element_type=jnp.float32)
    tsrc_ref[...] = _pack_pair(ms[:, :d], ms[:, d:])
    tdst_ref[...] = _pack_pair(md[:, :d], md[:, d:])

    @pl.when(pl.program_id(0) == 0)
    def _():
        s32_ref[...] = (
            jnp.dot(st_ref[...], kst_ref[...],
                    preferred_element_type=jnp.float32)
            + b_ref[...]
        )


_CHG = 128


def _gather_body(tsrc_hbm, tdst_hbm, src_hbm, dst_hbm, ps_hbm, pd_hbm,
                 is0, is1, id0, id1, ra0, ra1, rb0, rb1,
                 si0, si1, sg0, sg1, sw0, sw1):
    e = src_hbm.shape[0]
    nch = e // _CHG
    cpw = nch // _NW
    extra = nch - _NW * cpw
    wid = lax.axis_index("s") * _NC + lax.axis_index("c")
    isb, idb = (is0, is1), (id0, id1)
    rab, rbb = (ra0, ra1), (rb0, rb1)
    sib, sgb, swb = (si0, si1), (sg0, sg1), (sw0, sw1)

    def fire_idx(chunk, b):
        base = chunk * _CHG
        pltpu.async_copy(src_hbm.at[pl.ds(base, _CHG)], isb[b], sib[b])
        pltpu.async_copy(dst_hbm.at[pl.ds(base, _CHG)], idb[b], sib[b])

    def drain_idx(b):
        pltpu.make_async_copy(src_hbm.at[pl.ds(0, _CHG)], isb[b], sib[b]).wait()
        pltpu.make_async_copy(dst_hbm.at[pl.ds(0, _CHG)], idb[b], sib[b]).wait()

    def fire_gather(b):
        pltpu.async_copy(tsrc_hbm.at[isb[b]], rab[b], sgb[b])
        pltpu.async_copy(tdst_hbm.at[idb[b]], rbb[b], sgb[b])

    def drain_gather(b):
        pltpu.make_async_copy(tsrc_hbm.at[pl.ds(0, _CHG)], rab[b], sgb[b]).wait()
        pltpu.make_async_copy(tsrc_hbm.at[pl.ds(0, _CHG)], rbb[b], sgb[b]).wait()

    def fire_w(chunk, b):
        pltpu.async_copy(rab[b], ps_hbm.at[pl.ds(chunk * _CHG, _CHG)], swb[b])
        pltpu.async_copy(rbb[b], pd_hbm.at[pl.ds(chunk * _CHG, _CHG)], swb[b])

    def drain_w(b):
        pltpu.make_async_copy(rab[b], ps_hbm.at[pl.ds(0, _CHG)], swb[b]).wait()
        pltpu.make_async_copy(rbb[b], pd_hbm.at[pl.ds(0, _CHG)], swb[b]).wait()

    first = wid * cpw
    fire_idx(first, 0)
    drain_idx(0)
    fire_gather(0)
    fire_idx(first + 1, 1)

    def guard(cond, fn):
        if isinstance(cond, bool):
            if cond:
                fn()
        else:
            pl.when(cond)(fn)

    def step(i, b):
        guard(i + 1 < cpw, lambda: drain_idx(1 - b))
        guard(i >= 1, lambda: drain_w(1 - b))
        guard(i + 1 < cpw, lambda: fire_gather(1 - b))
        drain_gather(b)
        guard(i + 2 < cpw, lambda: fire_idx(first + i + 2, b))
        fire_w(first + i, b)

    def pair(p, carry):
        step(2 * p, 0)
        step(2 * p + 1, 1)
        return carry

    lax.fori_loop(0, cpw // 2, pair, 0)
    if cpw % 2:
        step(cpw - 1, 0)
    drain_w((cpw - 1) % 2)

    @pl.when(wid < extra)
    def _():
        chunk = _NW * cpw + wid
        fire_idx(chunk, 0)
        drain_idx(0)
        fire_gather(0)
        drain_gather(0)
        fire_w(chunk, 0)
        drain_w(0)


def _unpack_pair(w):
    lo = lax.bitcast_convert_type(w << 16, jnp.float32)
    hi = lax.bitcast_convert_type(w & jnp.uint32(0xFFFF0000), jnp.float32)
    return lo, hi


def _edge_body(ps_ref, pd_ref, bond_ref, bg_ref, s32_ref, kb_ref, out_ref):
    ws = lax.bitcast_convert_type(ps_ref[...], jnp.uint32)
    wd = lax.bitcast_convert_type(pd_ref[...], jnp.uint32)
    ts_s, tg_s = _unpack_pair(ws)
    ts_d, tg_d = _unpack_pair(wd)
    ts_pre = ts_s + ts_d
    tg_pre = tg_s + tg_d
    bond = bond_ref[...]
    row = bg_ref[0]
    ng = s32_ref.shape[0]
    onehot_t = (lax.broadcasted_iota(jnp.int32, (ng, row.shape[1]), 0)
                == row).astype(jnp.float32)
    contrib = lax.dot_general(onehot_t, s32_ref[...],
                              (((0,), (0,)), ((), ())),
                              preferred_element_type=jnp.float32)
    bk = jnp.dot(bond, kb_ref[...], preferred_element_type=jnp.float32)
    d = out_ref.shape[1]
    ts = ts_pre + bk[:, :d] + contrib[:, :d]
    tg = tg_pre + bk[:, d:] + contrib[:, d:]
    sig = 1.0 / (1.0 + jnp.exp(-ts))
    out_ref[...] = sig * _softplus(tg)


_TR = 512
_TRP = _TR + 16

_CHD = 64

def _scatter_body(tr0_hbm, src_hbm, agg_hbm, acc_sh, vb0, vb1,
                  ib0, ib1, ibuf2, zbuf, sp0, sp1):
    e = src_hbm.shape[0]
    nt = acc_sh.shape[0]
    nch = e // _CHD
    cpw = nch // _NW
    extra = nch - _NW * cpw
    da = acc_sh.shape[2]
    zr = zbuf.shape[0]
    cid = lax.axis_index("c")
    sid = lax.axis_index("s")
    wid = sid * _NC + cid
    vbb, ibb, spb = (vb0, vb1), (ib0, ib1), (sp0, sp1)

    def zero_row(r, carry):
        for j in range(da // _L):
            zbuf[r, pl.ds(j * _L, _L)] = jnp.zeros((_L,), jnp.float32)
        return carry

    lax.fori_loop(0, zr, zero_row, 0)
    for k in range(_TRP // zr):
        pltpu.sync_copy(zbuf, acc_sh.at[sid, pl.ds(k * zr, zr)])
        @pl.when(sid < nt - _NS)
        def _():
            pltpu.sync_copy(zbuf, acc_sh.at[_NS + sid, pl.ds(k * zr, zr)])
    plsc.subcore_barrier()

    def fire_p(cidx, b):
        base = cidx * _CHD
        pltpu.async_copy(src_hbm.at[pl.ds(base, _CHD)], ibb[b], spb[b])
        pltpu.async_copy(tr0_hbm.at[pl.ds(base, _CHD)], vbb[b], spb[b])

    def drain_p(b):
        pltpu.make_async_copy(src_hbm.at[pl.ds(0, _CHD)], ibb[b], spb[b]).wait()
        pltpu.make_async_copy(tr0_hbm.at[pl.ds(0, _CHD)], vbb[b],
                              spb[b]).wait()

    def scatter_chunk(b):
        ibuf = ibb[b]
        t_lo = ibuf[pl.ds(0, _L)][0] // _TR
        t_hi = ibuf[pl.ds(_CHD - _L, _L)][_L - 1] // _TR

        def tbody(t, carry):
            shift = t * _TR - 8
            for j in range(_CHD // _L):
                sl = pl.ds(j * _L, _L)
                ibuf2[sl] = jnp.clip(ibuf[sl] - shift, 0, _TR + 8)
            pltpu.sync_copy(vbb[b], acc_sh.at[t].at[ibuf2], add=True)
            return carry

        lax.fori_loop(t_lo, t_hi + 1, tbody, 0)

    first = wid * cpw
    fire_p(first, 0)
    fire_p(first + 1, 1)

    def step(i, b):
        drain_p(b)
        scatter_chunk(b)
        @pl.when(i + 2 < cpw)
        def _():
            fire_p(first + i + 2, b)

    def pair(p, carry):
        step(2 * p, 0)
        step(2 * p + 1, 1)
        return carry

    lax.fori_loop(0, cpw // 2, pair, 0)

    @pl.when(wid < extra)
    def _():
        chunk = _NW * cpw + wid
        fire_p(chunk, 0)
        drain_p(0)
        scatter_chunk(0)

    plsc.subcore_barrier()

    pltpu.sync_copy(acc_sh.at[sid, pl.ds(8, _TR)],
                    agg_hbm.at[pl.ds(cid * nt * _TR + sid * _TR, _TR)])
    @pl.when(sid < nt - _NS)
    def _():
        pltpu.sync_copy(acc_sh.at[_NS + sid, pl.ds(8, _TR)],
                        agg_hbm.at[pl.ds(cid * nt * _TR + (_NS + sid) * _TR, _TR)])


def _out_body(x_ref, a0_ref, a1_ref, a2_ref, a3_ref, out_ref):
    t = (x_ref[...] + a0_ref[...] + a1_ref[...]
         + a2_ref[...] + a3_ref[...])
    out_ref[...] = _softplus(t)


def kernel(atom_features, bond_features, state_attrs, pair_indices,
           atom_graph_indices, bond_graph_indices,
           kernel_s, bias_s, kernel_g, bias_g):
    del atom_graph_indices
    n, da = atom_features.shape
    e, de = bond_features.shape
    ng, dst_dim = state_attrs.shape
    dcat = 2 * da

    kk = jnp.concatenate([kernel_s, kernel_g], axis=1)
    k_src = kk[:da]
    k_dst = kk[da:2 * da]
    k_state = kk[2 * da:2 * da + dst_dim]
    k_bond = kk[2 * da + dst_dim:]
    bias = jnp.concatenate([bias_s, bias_g]).reshape(1, dcat)

    src = pair_indices[:, 0]
    dst = pair_indices[:, 1]

    nb = 10
    bn = n // nb
    tsrc, tdst, s32 = pl.pallas_call(
        _proj_body,
        grid=(nb,),
        in_specs=[
            pl.BlockSpec((bn, da), lambda i: (i, 0)),
            pl.BlockSpec((da, dcat), lambda i: (0, 0)),
            pl.BlockSpec((da, dcat), lambda i: (0, 0)),
            pl.BlockSpec((ng, dst_dim), lambda i: (0, 0)),
            pl.BlockSpec((dst_dim, dcat), lambda i: (0, 0)),
            pl.BlockSpec((1, dcat), lambda i: (0, 0)),
        ],
        out_specs=[
            pl.BlockSpec((bn, da), lambda i: (i, 0)),
            pl.BlockSpec((bn, da), lambda i: (i, 0)),
            pl.BlockSpec((ng, dcat), lambda i: (0, 0)),
        ],
        out_shape=[
            jax.ShapeDtypeStruct((n, da), jnp.int32),
            jax.ShapeDtypeStruct((n, da), jnp.int32),
            jax.ShapeDtypeStruct((ng, dcat), jnp.float32),
        ],
    )(atom_features, k_src, k_dst, state_attrs, k_state, bias)

    mesh = plsc.VectorSubcoreMesh(core_axis_name="c", subcore_axis_name="s")
    eh = e // 2
    be = 640
    nbe_h = eh // be
    bg3 = bond_graph_indices.reshape(e // be, 1, be)
    pre_halves = []
    for h in range(2):
        pre_s, pre_d = pl.kernel(
            _gather_body,
            mesh=mesh,
            out_type=[
                jax.ShapeDtypeStruct((eh, da), jnp.int32),
                jax.ShapeDtypeStruct((eh, da), jnp.int32),
            ],
            scratch_types=(
                [pltpu.VMEM((_CHG,), jnp.int32)] * 4
                + [pltpu.VMEM((_CHG, da), jnp.int32)] * 4
                + [pltpu.SemaphoreType.DMA] * 6
            ),
        )(tsrc, tdst, src[h * eh:(h + 1) * eh], dst[h * eh:(h + 1) * eh])
        pre_halves.append((pre_s, pre_d))

    trans_halves = []
    for h in range(2):
        off = h * nbe_h
        trans_halves.append(pl.pallas_call(
            _edge_body,
            grid=(nbe_h,),
            in_specs=[
                pl.BlockSpec((be, da), lambda i: (i, 0)),
                pl.BlockSpec((be, da), lambda i: (i, 0)),
                pl.BlockSpec((be, de), lambda i, off=off: (i + off, 0)),
                pl.BlockSpec((1, 1, be), lambda i, off=off: (i + off, 0, 0)),
                pl.BlockSpec((ng, dcat), lambda i: (0, 0)),
                pl.BlockSpec((de, dcat), lambda i: (0, 0)),
            ],
            out_specs=pl.BlockSpec((be, da), lambda i: (i, 0)),
            out_shape=jax.ShapeDtypeStruct((eh, da), jnp.float32),
        )(pre_halves[h][0], pre_halves[h][1], bond_features, bg3, s32, k_bond))

    nt = (n + _TR - 1) // _TR
    n_pad = nt * _TR
    aggs = []
    for h in range(2):
        aggs.append(pl.kernel(
            _scatter_body,
            mesh=mesh,
            out_type=jax.ShapeDtypeStruct((_NC * n_pad, da), jnp.float32),
            scratch_types=[
                pltpu.VMEM_SHARED((nt, _TRP, da), jnp.float32),
                pltpu.VMEM((_CHD, da), jnp.float32),
                pltpu.VMEM((_CHD, da), jnp.float32),
                pltpu.VMEM((_CHD,), jnp.int32),
                pltpu.VMEM((_CHD,), jnp.int32),
                pltpu.VMEM((_CHD,), jnp.int32),
                pltpu.VMEM((_TRP // 6, da), jnp.float32),
                pltpu.SemaphoreType.DMA,
                pltpu.SemaphoreType.DMA,
            ],
        )(trans_halves[h], src[h * eh:(h + 1) * eh]))

    bn_e = 80
    nb_e = n // bn_e
    off = n_pad // bn_e
    out = pl.pallas_call(
        _out_body,
        grid=(nb_e,),
        in_specs=[
            pl.BlockSpec((bn_e, da), lambda i: (i, 0)),
            pl.BlockSpec((bn_e, da), lambda i: (i, 0)),
            pl.BlockSpec((bn_e, da), lambda i: (i + off, 0)),
            pl.BlockSpec((bn_e, da), lambda i: (i, 0)),
            pl.BlockSpec((bn_e, da), lambda i: (i + off, 0)),
        ],
        out_specs=pl.BlockSpec((bn_e, da), lambda i: (i, 0)),
        out_shape=jax.ShapeDtypeStruct((n, da), jnp.float32),
    )(atom_features, aggs[0], aggs[0], aggs[1], aggs[1])
    return out
